# baseline (speedup 1.0000x reference)
import math

import jax
import jax.numpy as jnp
from jax.experimental import pallas as pl

N = 10000
D = 256
H = 256
G = 64
OUT = 2
R = 8
B = N // R
NEG_INF = float("-inf")


def _seg_scan(vals, ids, nrows, nsteps):
    f32 = jnp.float32
    for k in range(nsteps):
        s = 1 << k
        v_sh = jnp.concatenate(
            [jnp.full((s, vals.shape[1]), NEG_INF, dtype=f32),
             jax.lax.slice(vals, (0, 0), (nrows - s, vals.shape[1]))], axis=0)
        i_sh = jnp.concatenate(
            [jnp.full((s, 1), -1, dtype=jnp.int32),
             jax.lax.slice(ids, (0, 0), (nrows - s, 1))], axis=0)
        same = i_sh == ids
        vals = jnp.maximum(vals, jnp.where(same, v_sh, NEG_INF))
    return vals


def _fused_kernel(x_ref, batch_col_ref, batch_row_ref, tail_row_ref,
                  tail_col_ref,
                  w1t_ref, b1_ref, wc1t_ref, bc1_ref,
                  wc2t_ref, bc2_ref, wc3t_ref, bc3_ref,
                  out_ref):
    f32 = jnp.float32

    h = jnp.dot(x_ref[...], w1t_ref[...], preferred_element_type=f32)
    h = h + b1_ref[...]

    batch_col = batch_col_ref[...]
    batch_row = batch_row_ref[...]
    tail_row = tail_row_ref[...]

    seg_iota = jax.lax.broadcasted_iota(jnp.int32, (G, 1), 0)
    eq = (batch_row == seg_iota).astype(f32)
    le = (batch_row <= seg_iota).astype(f32)

    counts = jnp.sum(eq, axis=1, keepdims=True)
    ends = jnp.sum(le, axis=1, keepdims=True).astype(jnp.int32) - 1

    x_sum = jnp.dot(eq, h, preferred_element_type=f32)

    m = _seg_scan(h, batch_col, N, int(math.log2(R)))
    m3 = jnp.reshape(m, (B, R, H))
    tails = jnp.reshape(jax.lax.slice(m3, (0, R - 1, 0), (B, R, H)), (B, H))
    tail_col = tail_col_ref[...]
    tails = _seg_scan(tails, tail_col, B, int(math.ceil(math.log2(B))))

    col_iota = jax.lax.broadcasted_iota(jnp.int32, (1, N), 1)
    sel = ((col_iota == ends) & (counts > 0.0)).astype(f32)
    g_end = jnp.dot(sel, m, preferred_element_type=f32)

    eq_t = (tail_row == seg_iota).astype(f32)
    le_t = (tail_row <= seg_iota).astype(f32)
    counts_t = jnp.sum(eq_t, axis=1, keepdims=True)
    ends_t = jnp.sum(le_t, axis=1, keepdims=True).astype(jnp.int32) - 1
    colb_iota = jax.lax.broadcasted_iota(jnp.int32, (1, B), 1)
    sel_t = ((colb_iota == ends_t) & (counts_t > 0.0)).astype(f32)
    g_tail = jnp.dot(sel_t, tails, preferred_element_type=f32)
    g_tail = jnp.where(counts_t > 0.0, g_tail, NEG_INF)

    x_max = jnp.where(counts > 0.0, jnp.maximum(g_end, g_tail), NEG_INF)

    x_mean = x_sum / jnp.maximum(counts, 1.0)

    x_global = jnp.concatenate([x_mean, x_max, x_sum], axis=1)

    z = jnp.dot(x_global, wc1t_ref[...], preferred_element_type=f32)
    z = jnp.maximum(z + bc1_ref[...], 0.0)
    z = jnp.dot(z, wc2t_ref[...], preferred_element_type=f32)
    z = jnp.maximum(z + bc2_ref[...], 0.0)
    z = jnp.dot(z, wc3t_ref[...], preferred_element_type=f32)
    out_ref[...] = z + bc3_ref[...]


@jax.jit
def _run(x, batch, W1, b1, Wc1, bc1, Wc2, bc2, Wc3, bc3):
    batch_col = batch.reshape(N, 1)
    batch_row = batch.reshape(1, N)
    tail_row = batch[R - 1::R].reshape(1, B)
    tail_col = batch[R - 1::R].reshape(B, 1)
    wc3t_pad = jnp.zeros((H // 2, 128), jnp.float32).at[:, :OUT].set(Wc3.T)
    bc3_pad = jnp.zeros((1, 128), jnp.float32).at[:, :OUT].set(bc3)
    out = pl.pallas_call(
        _fused_kernel,
        out_shape=jax.ShapeDtypeStruct((G, 128), jnp.float32),
    )(x, batch_col, batch_row, tail_row, tail_col,
      W1.T, b1.reshape(1, H), Wc1.T, bc1.reshape(1, H),
      Wc2.T, bc2.reshape(1, H // 2), wc3t_pad, bc3_pad)
    return out[:, :OUT]


def kernel(x, edge_index, batch, W1, b1, Wc1, bc1, Wc2, bc2, Wc3, bc3):
    del edge_index
    return _run(x, batch, W1, b1, Wc1, bc1, Wc2, bc2, Wc3, bc3)


# speedup vs baseline: 1.2964x; 1.2964x over previous
import math

import jax
import jax.numpy as jnp
from jax.experimental import pallas as pl

N = 10000
D = 256
H = 256
G = 64
OUT = 2
R = 8
B = N // R
NEG_INF = float("-inf")


def _dot_rt(a, b):
    return jax.lax.dot_general(a, b, (((1,), (1,)), ((), ())),
                               preferred_element_type=jnp.float32)


def _seg_scan(vals, ids, nrows, nsteps):
    f32 = jnp.float32
    for k in range(nsteps):
        s = 1 << k
        v_sh = jnp.concatenate(
            [jnp.full((s, vals.shape[1]), NEG_INF, dtype=f32),
             jax.lax.slice(vals, (0, 0), (nrows - s, vals.shape[1]))], axis=0)
        i_sh = jnp.concatenate(
            [jnp.full((s, 1), -1, dtype=jnp.int32),
             jax.lax.slice(ids, (0, 0), (nrows - s, 1))], axis=0)
        same = i_sh == ids
        vals = jnp.maximum(vals, jnp.where(same, v_sh, NEG_INF))
    return vals


def _fused_kernel(x_ref, batch_col_ref, batch_row_ref,
                  w1_ref, b1_ref, wc1_ref, bc1_ref,
                  wc2_ref, bc2_ref, wc3_ref, bc3_ref,
                  out_ref):
    f32 = jnp.float32

    h = _dot_rt(x_ref[...], w1_ref[...]) + jnp.reshape(b1_ref[...], (1, H))

    batch_col = batch_col_ref[...]
    batch_row = batch_row_ref[...]

    seg_iota = jax.lax.broadcasted_iota(jnp.int32, (G, 1), 0)
    eq = (batch_row == seg_iota).astype(f32)
    le = (batch_row <= seg_iota).astype(f32)

    counts = jnp.sum(eq, axis=1, keepdims=True)
    ends = jnp.sum(le, axis=1, keepdims=True).astype(jnp.int32) - 1

    x_sum = jnp.dot(eq, h, preferred_element_type=f32)

    m = _seg_scan(h, batch_col, N, int(math.log2(R)))
    m3 = jnp.reshape(m, (B, R, H))
    tails = jnp.reshape(jax.lax.slice(m3, (0, R - 1, 0), (B, R, H)), (B, H))
    b3 = jnp.reshape(batch_col, (B, R, 1))
    tail_col = jnp.reshape(jax.lax.slice(b3, (0, R - 1, 0), (B, R, 1)), (B, 1))
    tails = _seg_scan(tails, tail_col, B, int(math.ceil(math.log2(B))))

    col_iota = jax.lax.broadcasted_iota(jnp.int32, (1, N), 1)
    sel = ((col_iota == ends) & (counts > 0.0)).astype(f32)
    g_end = jnp.dot(sel, m, preferred_element_type=f32)

    tmask = (col_iota % R == (R - 1)).astype(f32)
    counts_t = jnp.sum(eq * tmask, axis=1, keepdims=True)
    ends_t = jnp.sum(le * tmask, axis=1, keepdims=True).astype(jnp.int32) - 1
    colb_iota = jax.lax.broadcasted_iota(jnp.int32, (1, B), 1)
    sel_t = ((colb_iota == ends_t) & (counts_t > 0.0)).astype(f32)
    g_tail = jnp.dot(sel_t, tails, preferred_element_type=f32)
    g_tail = jnp.where(counts_t > 0.0, g_tail, NEG_INF)

    x_max = jnp.where(counts > 0.0, jnp.maximum(g_end, g_tail), NEG_INF)

    x_mean = x_sum / jnp.maximum(counts, 1.0)

    x_global = jnp.concatenate([x_mean, x_max, x_sum], axis=1)

    z = _dot_rt(x_global, wc1_ref[...]) + jnp.reshape(bc1_ref[...], (1, H))
    z = jnp.maximum(z, 0.0)
    z = _dot_rt(z, wc2_ref[...]) + jnp.reshape(bc2_ref[...], (1, H // 2))
    z = jnp.maximum(z, 0.0)
    z = _dot_rt(z, wc3_ref[...]) + jnp.reshape(bc3_ref[...], (1, OUT))
    out_ref[...] = z


@jax.jit
def _run(x, batch, W1, b1, Wc1, bc1, Wc2, bc2, Wc3, bc3):
    batch_col = batch.reshape(N, 1)
    batch_row = batch.reshape(1, N)
    return pl.pallas_call(
        _fused_kernel,
        out_shape=jax.ShapeDtypeStruct((G, OUT), jnp.float32),
    )(x, batch_col, batch_row,
      W1, b1, Wc1, bc1, Wc2, bc2, Wc3, bc3)


def kernel(x, edge_index, batch, W1, b1, Wc1, bc1, Wc2, bc2, Wc3, bc3):
    del edge_index
    return _run(x, batch, W1, b1, Wc1, bc1, Wc2, bc2, Wc3, bc3)


# speedup vs baseline: 1.3446x; 1.0372x over previous
import math

import jax
import jax.numpy as jnp
from jax.experimental import pallas as pl

N = 10000
D = 256
H = 256
G = 64
OUT = 2
R = 8
B = N // R
NEG_INF = float("-inf")


def _dot_rt(a, b):
    return jax.lax.dot_general(a, b, (((1,), (1,)), ((), ())),
                               preferred_element_type=jnp.float32)


def _seg_scan(vals, ids, nrows, nsteps):
    f32 = jnp.float32
    for k in range(nsteps):
        s = 1 << k
        v_sh = jnp.concatenate(
            [jnp.full((s, vals.shape[1]), NEG_INF, dtype=f32),
             jax.lax.slice(vals, (0, 0), (nrows - s, vals.shape[1]))], axis=0)
        i_sh = jnp.concatenate(
            [jnp.full((s, 1), -1, dtype=jnp.int32),
             jax.lax.slice(ids, (0, 0), (nrows - s, 1))], axis=0)
        same = i_sh == ids
        vals = jnp.maximum(vals, jnp.where(same, v_sh, NEG_INF))
    return vals


def _fused_kernel(x_ref, batch_col_ref, batch_row_ref,
                  w1_ref, b1_ref, wc1_ref, bc1_ref,
                  wc2_ref, bc2_ref, wc3_ref, bc3_ref,
                  out_ref):
    f32 = jnp.float32

    h = _dot_rt(x_ref[...], w1_ref[...]) + jnp.reshape(b1_ref[...], (1, H))

    batch_col = batch_col_ref[...]
    batch_row = batch_row_ref[...]

    seg_iota = jax.lax.broadcasted_iota(jnp.int32, (G, 1), 0)
    eq = (batch_row == seg_iota).astype(f32)
    le = (batch_row <= seg_iota).astype(f32)

    counts = jnp.sum(eq, axis=1, keepdims=True)
    ends = jnp.sum(le, axis=1, keepdims=True).astype(jnp.int32) - 1

    x_sum = jnp.dot(eq, h, preferred_element_type=f32)

    m = _seg_scan(h, batch_col, N, int(math.ceil(math.log2(N))))

    col_iota = jax.lax.broadcasted_iota(jnp.int32, (1, N), 1)
    sel = ((col_iota == ends) & (counts > 0.0)).astype(f32)
    g_end = jnp.dot(sel, m, preferred_element_type=f32)

    x_max = jnp.where(counts > 0.0, g_end, NEG_INF)

    x_mean = x_sum / jnp.maximum(counts, 1.0)

    x_global = jnp.concatenate([x_mean, x_max, x_sum], axis=1)

    z = _dot_rt(x_global, wc1_ref[...]) + jnp.reshape(bc1_ref[...], (1, H))
    z = jnp.maximum(z, 0.0)
    z = _dot_rt(z, wc2_ref[...]) + jnp.reshape(bc2_ref[...], (1, H // 2))
    z = jnp.maximum(z, 0.0)
    z = _dot_rt(z, wc3_ref[...]) + jnp.reshape(bc3_ref[...], (1, OUT))
    out_ref[...] = z


@jax.jit
def _run(x, batch, W1, b1, Wc1, bc1, Wc2, bc2, Wc3, bc3):
    batch_col = batch.reshape(N, 1)
    batch_row = batch.reshape(1, N)
    return pl.pallas_call(
        _fused_kernel,
        out_shape=jax.ShapeDtypeStruct((G, OUT), jnp.float32),
    )(x, batch_col, batch_row,
      W1, b1, Wc1, bc1, Wc2, bc2, Wc3, bc3)


def kernel(x, edge_index, batch, W1, b1, Wc1, bc1, Wc2, bc2, Wc3, bc3):
    del edge_index
    return _run(x, batch, W1, b1, Wc1, bc1, Wc2, bc2, Wc3, bc3)
